# baseline (speedup 1.0000x reference)
import functools

import jax
import jax.numpy as jnp
from jax import lax
from jax.experimental import pallas as pl
from jax.experimental.pallas import tpu as pltpu
from jax.experimental.pallas import tpu_sc as plsc

N = 10000
E = 160000
NF = 128
OF = 64
EF = 16
H = 64

NW = 32
CHUNK = 128
EPW = 5120
E_PAD = NW * EPW
N_CHUNKS = EPW // CHUNK
BLK = 512

def _mesh():
    return plsc.VectorSubcoreMesh(core_axis_name="c", subcore_axis_name="s")


@functools.cache
def _sc_gather_fn():
    @functools.partial(
        pl.kernel,
        out_type=jax.ShapeDtypeStruct((E_PAD, NF), jnp.float32),
        mesh=_mesh(),
        scratch_types=[
            pltpu.VMEM((CHUNK,), jnp.int32),
            pltpu.VMEM((CHUNK, NF), jnp.float32),
            pltpu.SemaphoreType.DMA,
        ],
    )
    def gather(nfeat_hbm, src_hbm, out_hbm, idx_v, rows_v, sem):
        wid = lax.axis_index("s") * 2 + lax.axis_index("c")
        base = wid * EPW

        def body(j, carry):
            off = base + j * CHUNK
            pltpu.sync_copy(src_hbm.at[pl.ds(off, CHUNK)], idx_v)
            pltpu.async_copy(nfeat_hbm.at[idx_v], rows_v, sem).wait()
            pltpu.sync_copy(rows_v, out_hbm.at[pl.ds(off, CHUNK)])
            return carry

        lax.fori_loop(0, N_CHUNKS, body, 0)

    return gather


def _sc_gather(nfeat, src_pad):
    return _sc_gather_fn()(nfeat, src_pad)


N_PAD = 10240
_RPT = N_PAD // 16
_RCHUNK = 128
_RN = _RPT // _RCHUNK


@functools.cache
def _sc_scatter_fn():
    @functools.partial(
        pl.kernel,
        out_type=jax.ShapeDtypeStruct((2, N_PAD, 128), jnp.float32),
        mesh=_mesh(),
        scratch_types=[
            pltpu.VMEM_SHARED((N_PAD, 128), jnp.float32),
            pltpu.VMEM((CHUNK, 128), jnp.float32),
            pltpu.VMEM((CHUNK,), jnp.int32),
        ],
    )
    def scatter(msg_hbm, dst_hbm, zeros_hbm, out_hbm, acc_sh, mrows_v, idx_v):
        cid = lax.axis_index("c")
        sid = lax.axis_index("s")
        wid = sid * 2 + cid
        r0 = sid * _RPT

        def zbody(k, carry):
            rr = r0 + k * _RCHUNK
            pltpu.sync_copy(zeros_hbm.at[pl.ds(rr, _RCHUNK)], mrows_v.at[pl.ds(0, _RCHUNK)])
            pltpu.sync_copy(mrows_v.at[pl.ds(0, _RCHUNK)], acc_sh.at[pl.ds(rr, _RCHUNK)])
            return carry

        lax.fori_loop(0, _RN, zbody, 0)
        plsc.subcore_barrier()

        base = wid * EPW

        def body(j, carry):
            off = base + j * CHUNK
            pltpu.sync_copy(dst_hbm.at[pl.ds(off, CHUNK)], idx_v)
            pltpu.sync_copy(msg_hbm.at[pl.ds(off, CHUNK)], mrows_v)
            pltpu.sync_copy(mrows_v, acc_sh.at[idx_v], add=True)
            return carry

        lax.fori_loop(0, N_CHUNKS, body, 0)
        plsc.subcore_barrier()

        def obody(k, carry):
            rr = r0 + k * _RCHUNK
            pltpu.sync_copy(acc_sh.at[pl.ds(rr, _RCHUNK)], mrows_v.at[pl.ds(0, _RCHUNK)])
            pltpu.sync_copy(mrows_v.at[pl.ds(0, _RCHUNK)], out_hbm.at[cid, pl.ds(rr, _RCHUNK)])
            return carry

        lax.fori_loop(0, _RN, obody, 0)

    return scatter


def _sc_scatter(msg, dst_pad, zeros):
    return _sc_scatter_fn()(msg, dst_pad, zeros)


def _msg_body(srch_ref, ef_ref, acat_ref, out_ref):
    sf = jnp.dot(srch_ref[...], acat_ref[...], preferred_element_type=jnp.float32)
    ef = ef_ref[...]
    acc = ef[:, 0:1] * sf[:, 0:OF]
    for f in range(1, EF + 1):
        acc = acc + ef[:, f : f + 1] * sf[:, f * OF : (f + 1) * OF]
    out_ref[...] = jnp.concatenate([acc, jnp.zeros_like(acc)], axis=1)


def _tc_msg(src_h, efeat_ext, acat):
    grid = E_PAD // BLK
    return pl.pallas_call(
        _msg_body,
        grid=(grid,),
        in_specs=[
            pl.BlockSpec((BLK, NF), lambda i: (i, 0)),
            pl.BlockSpec((BLK, 32), lambda i: (i, 0)),
            pl.BlockSpec((NF, (EF + 1) * OF), lambda i: (0, 0)),
        ],
        out_specs=pl.BlockSpec((BLK, 2 * OF), lambda i: (i, 0)),
        out_shape=jax.ShapeDtypeStruct((E_PAD, 2 * OF), jnp.float32),
    )(src_h, efeat_ext, acat)


def _head_body(parts_ref, eps_ref, w1t_ref, b1_ref, g1_ref, be1_ref,
               w2t_ref, b2_ref, g2_ref, be2_ref, out_ref):
    agg = (1.0 + eps_ref[0, 0]) * (parts_ref[0] + parts_ref[1])
    x = jnp.dot(agg, w1t_ref[...], preferred_element_type=jnp.float32) + b1_ref[...]
    m = jnp.mean(x, axis=0, keepdims=True)
    v = jnp.mean((x - m) ** 2, axis=0, keepdims=True)
    x = g1_ref[...] * (x - m) * lax.rsqrt(v + 1e-5) + be1_ref[...]
    x = jnp.maximum(x, 0.0)
    x = jnp.dot(x, w2t_ref[...], preferred_element_type=jnp.float32) + b2_ref[...]
    m2 = jnp.mean(x, axis=0, keepdims=True)
    v2 = jnp.mean((x - m2) ** 2, axis=0, keepdims=True)
    x = g2_ref[...] * (x - m2) * lax.rsqrt(v2 + 1e-5) + be2_ref[...]
    out_ref[...] = jnp.where(x >= 0.0, x, 0.01 * x)


def _tc_head(parts, eps, w1t, b1, g1, be1, w2t, b2, g2, be2):
    return pl.pallas_call(
        _head_body,
        in_specs=[
            pl.BlockSpec(memory_space=pltpu.VMEM),
            pl.BlockSpec(memory_space=pltpu.SMEM),
        ] + [pl.BlockSpec(memory_space=pltpu.VMEM)] * 8,
        out_specs=pl.BlockSpec(memory_space=pltpu.VMEM),
        out_shape=jax.ShapeDtypeStruct((N, OF), jnp.float32),
    )(parts, eps, w1t, b1, g1, be1, w2t, b2, g2, be2)


def kernel(nfeat, efeat, edge_index, eps, A, Ab, W1, b1, g1, be1, W2, b2, g2, be2):
    src = edge_index[0]
    dst = edge_index[1]
    pad = E_PAD - E
    src_pad = jnp.concatenate([src, jnp.zeros((pad,), jnp.int32)])
    dst_pad = jnp.concatenate([dst, jnp.zeros((pad,), jnp.int32)])
    ef_ext = jnp.concatenate([efeat, jnp.ones((E, 1), jnp.float32)], axis=1)
    ef_ext = jnp.pad(ef_ext, ((0, pad), (0, 32 - (EF + 1))))
    A3 = A.reshape(EF, NF, OF)
    acat = jnp.concatenate(
        [A3.transpose(1, 0, 2).reshape(NF, EF * OF), Ab.reshape(NF, OF)], axis=1
    )

    src_h = _sc_gather(nfeat, src_pad)
    msg = _tc_msg(src_h, ef_ext, acat)
    zeros = jnp.zeros((N_PAD, 128), jnp.float32)
    parts = _sc_scatter(msg, dst_pad, zeros)[:, :N, :OF]
    return _tc_head(
        parts,
        eps.reshape(1, 1),
        W1.T, b1.reshape(1, H), g1.reshape(1, H), be1.reshape(1, H),
        W2.T, b2.reshape(1, OF), g2.reshape(1, OF), be2.reshape(1, OF),
    )


# speedup vs baseline: 1.0105x; 1.0105x over previous
import functools

import jax
import jax.numpy as jnp
from jax import lax
from jax.experimental import pallas as pl
from jax.experimental.pallas import tpu as pltpu
from jax.experimental.pallas import tpu_sc as plsc

N = 10000
E = 160000
NF = 128
OF = 64
EF = 16
H = 64

NW = 32
CHUNK = 128
EPW = 5120
E_PAD = NW * EPW
N_CHUNKS = EPW // CHUNK
BLK = 512

NACC = 5120
_RPT = NACC // 16
_RCHUNK = 64
_RN = _RPT // _RCHUNK


def _mesh():
    return plsc.VectorSubcoreMesh(core_axis_name="c", subcore_axis_name="s")


@functools.cache
def _sc_gather_fn():
    @functools.partial(
        pl.kernel,
        out_type=jax.ShapeDtypeStruct((E_PAD, NF), jnp.float32),
        mesh=_mesh(),
        scratch_types=[
            pltpu.VMEM((N_CHUNKS, CHUNK), jnp.int32),
            pltpu.VMEM((2, CHUNK, NF), jnp.float32),
            pltpu.SemaphoreType.DMA((2,)),
            pltpu.SemaphoreType.DMA((2,)),
        ],
    )
    def gather(nfeat_hbm, src_hbm, out_hbm, idx_all, rows_v, gsem, wsem):
        wid = lax.axis_index("s") * 2 + lax.axis_index("c")
        base = wid * EPW
        pltpu.sync_copy(src_hbm.at[wid], idx_all)
        pltpu.async_copy(nfeat_hbm.at[idx_all.at[0]], rows_v.at[0], gsem.at[0])

        def outer(t, carry):
            for b in range(2):
                j = 2 * t + b
                nb = 1 - b
                @pl.when(j < N_CHUNKS - 1)
                def _():
                    @pl.when(j >= 1)
                    def _():
                        pltpu.make_async_copy(
                            rows_v.at[nb],
                            out_hbm.at[pl.ds(base, CHUNK)],
                            wsem.at[nb],
                        ).wait()
                    pltpu.async_copy(
                        nfeat_hbm.at[idx_all.at[j + 1]], rows_v.at[nb], gsem.at[nb]
                    )

                pltpu.make_async_copy(
                    nfeat_hbm.at[idx_all.at[j]], rows_v.at[b], gsem.at[b]
                ).wait()
                pltpu.async_copy(
                    rows_v.at[b], out_hbm.at[pl.ds(base + j * CHUNK, CHUNK)], wsem.at[b]
                )
            return carry

        lax.fori_loop(0, N_CHUNKS // 2, outer, 0)
        for b in range(2):
            pltpu.make_async_copy(
                rows_v.at[b], out_hbm.at[pl.ds(base, CHUNK)], wsem.at[b]
            ).wait()

    return gather


def _sc_gather(nfeat, src3):
    return _sc_gather_fn()(nfeat, src3)


@functools.cache
def _sc_scatter_fn():
    @functools.partial(
        pl.kernel,
        out_type=jax.ShapeDtypeStruct((2, NACC, 128), jnp.float32),
        mesh=_mesh(),
        scratch_types=[
            pltpu.VMEM_SHARED((NACC, 128), jnp.float32),
            pltpu.VMEM((N_CHUNKS, CHUNK), jnp.int32),
            pltpu.VMEM((2, CHUNK, 128), jnp.float32),
            pltpu.SemaphoreType.DMA((2,)),
        ],
    )
    def scatter(msg_hbm, dst_hbm, zeros_hbm, out_hbm, acc_sh, idx_all, mrows_v, msem):
        cid = lax.axis_index("c")
        sid = lax.axis_index("s")
        wid = sid * 2 + cid
        r0 = sid * _RPT

        def zbody(k, carry):
            rr = r0 + k * _RCHUNK
            pltpu.sync_copy(zeros_hbm.at[pl.ds(rr, _RCHUNK)], mrows_v.at[0, pl.ds(0, _RCHUNK)])
            pltpu.sync_copy(mrows_v.at[0, pl.ds(0, _RCHUNK)], acc_sh.at[pl.ds(rr, _RCHUNK)])
            return carry

        lax.fori_loop(0, _RN, zbody, 0)
        pltpu.sync_copy(dst_hbm.at[wid], idx_all)
        plsc.subcore_barrier()

        base = wid * EPW
        pltpu.async_copy(msg_hbm.at[pl.ds(base, CHUNK)], mrows_v.at[0], msem.at[0])

        def body(t, carry):
            for b in range(2):
                j = 2 * t + b
                nb = 1 - b

                @pl.when(j < N_CHUNKS - 1)
                def _():
                    pltpu.async_copy(
                        msg_hbm.at[pl.ds(base + (j + 1) * CHUNK, CHUNK)],
                        mrows_v.at[nb],
                        msem.at[nb],
                    )

                pltpu.make_async_copy(
                    msg_hbm.at[pl.ds(base, CHUNK)], mrows_v.at[b], msem.at[b]
                ).wait()
                pltpu.sync_copy(mrows_v.at[b], acc_sh.at[idx_all.at[j]], add=True)
            return carry

        lax.fori_loop(0, N_CHUNKS // 2, body, 0)
        plsc.subcore_barrier()

        def obody(k, carry):
            rr = r0 + k * _RCHUNK
            pltpu.sync_copy(acc_sh.at[pl.ds(rr, _RCHUNK)], mrows_v.at[0, pl.ds(0, _RCHUNK)])
            pltpu.sync_copy(mrows_v.at[0, pl.ds(0, _RCHUNK)], out_hbm.at[cid, pl.ds(rr, _RCHUNK)])
            return carry

        lax.fori_loop(0, _RN, obody, 0)

    return scatter


def _sc_scatter(msg, dst3, zeros):
    return _sc_scatter_fn()(msg, dst3, zeros)


def _msg_body(srch_ref, ef_ref, acat_ref, out_ref):
    sf = jnp.dot(srch_ref[...], acat_ref[...], preferred_element_type=jnp.float32)
    ef = ef_ref[...]
    acc = ef[:, 0:1] * sf[:, 0:OF]
    for f in range(1, EF + 1):
        acc = acc + ef[:, f : f + 1] * sf[:, f * OF : (f + 1) * OF]
    out_ref[...] = jnp.concatenate(
        [acc * ef[:, EF + 1 : EF + 2], acc * ef[:, EF + 2 : EF + 3]], axis=1
    )


def _tc_msg(src_h, efeat_ext, acat):
    grid = E_PAD // BLK
    return pl.pallas_call(
        _msg_body,
        grid=(grid,),
        in_specs=[
            pl.BlockSpec((BLK, NF), lambda i: (i, 0)),
            pl.BlockSpec((BLK, 32), lambda i: (i, 0)),
            pl.BlockSpec((NF, (EF + 1) * OF), lambda i: (0, 0)),
        ],
        out_specs=pl.BlockSpec((BLK, 2 * OF), lambda i: (i, 0)),
        out_shape=jax.ShapeDtypeStruct((E_PAD, 2 * OF), jnp.float32),
    )(src_h, efeat_ext, acat)


def _head_body(parts_ref, eps_ref, w1t_ref, b1_ref, g1_ref, be1_ref,
               w2t_ref, b2_ref, g2_ref, be2_ref, out_ref):
    agg = (1.0 + eps_ref[0, 0]) * (parts_ref[0] + parts_ref[1])
    x = jnp.dot(agg, w1t_ref[...], preferred_element_type=jnp.float32) + b1_ref[...]
    m = jnp.mean(x, axis=0, keepdims=True)
    v = jnp.mean((x - m) ** 2, axis=0, keepdims=True)
    x = g1_ref[...] * (x - m) * lax.rsqrt(v + 1e-5) + be1_ref[...]
    x = jnp.maximum(x, 0.0)
    x = jnp.dot(x, w2t_ref[...], preferred_element_type=jnp.float32) + b2_ref[...]
    m2 = jnp.mean(x, axis=0, keepdims=True)
    v2 = jnp.mean((x - m2) ** 2, axis=0, keepdims=True)
    x = g2_ref[...] * (x - m2) * lax.rsqrt(v2 + 1e-5) + be2_ref[...]
    out_ref[...] = jnp.where(x >= 0.0, x, 0.01 * x)


def _tc_head(parts, eps, w1t, b1, g1, be1, w2t, b2, g2, be2):
    return pl.pallas_call(
        _head_body,
        in_specs=[
            pl.BlockSpec(memory_space=pltpu.VMEM),
            pl.BlockSpec(memory_space=pltpu.SMEM),
        ] + [pl.BlockSpec(memory_space=pltpu.VMEM)] * 8,
        out_specs=pl.BlockSpec(memory_space=pltpu.VMEM),
        out_shape=jax.ShapeDtypeStruct((N, OF), jnp.float32),
    )(parts, eps, w1t, b1, g1, be1, w2t, b2, g2, be2)


def kernel(nfeat, efeat, edge_index, eps, A, Ab, W1, b1, g1, be1, W2, b2, g2, be2):
    src = edge_index[0]
    dst = edge_index[1]
    pad = E_PAD - E
    src3 = jnp.concatenate([src, jnp.zeros((pad,), jnp.int32)]).reshape(
        NW, N_CHUNKS, CHUNK
    )
    dst_pad = jnp.concatenate([dst, jnp.zeros((pad,), jnp.int32)])
    dst3 = (dst_pad // 2).reshape(NW, N_CHUNKS, CHUNK)
    parity = (dst % 2).astype(jnp.float32)[:, None]
    ef_ext = jnp.concatenate(
        [efeat, jnp.ones((E, 1), jnp.float32), 1.0 - parity, parity], axis=1
    )
    ef_ext = jnp.pad(ef_ext, ((0, pad), (0, 32 - (EF + 3))))
    A3 = A.reshape(EF, NF, OF)
    acat = jnp.concatenate(
        [A3.transpose(1, 0, 2).reshape(NF, EF * OF), Ab.reshape(NF, OF)], axis=1
    )

    src_h = _sc_gather(nfeat, src3)
    msg = _tc_msg(src_h, ef_ext, acat)
    zeros = jnp.zeros((NACC, 128), jnp.float32)
    parts = _sc_scatter(msg, dst3, zeros)
    parts = parts.reshape(2, 2 * NACC, OF)[:, :N, :]
    return _tc_head(
        parts,
        eps.reshape(1, 1),
        W1.T, b1.reshape(1, H), g1.reshape(1, H), be1.reshape(1, H),
        W2.T, b2.reshape(1, OF), g2.reshape(1, OF), be2.reshape(1, OF),
    )


# speedup vs baseline: 1.0160x; 1.0055x over previous
import functools

import jax
import jax.numpy as jnp
from jax import lax
from jax.experimental import pallas as pl
from jax.experimental.pallas import tpu as pltpu
from jax.experimental.pallas import tpu_sc as plsc

N = 10000
E = 160000
NF = 128
OF = 64
EF = 16
H = 64

NW = 32
CHUNK = 128
EPW = 5120
E_PAD = NW * EPW
N_CHUNKS = EPW // CHUNK
BLK = 512

NACC = 5120
_RPT = NACC // 16
_RCHUNK = 64
_RN = _RPT // _RCHUNK


def _mesh():
    return plsc.VectorSubcoreMesh(core_axis_name="c", subcore_axis_name="s")


@functools.cache
def _sc_gather_fn():
    @functools.partial(
        pl.kernel,
        out_type=jax.ShapeDtypeStruct((E_PAD, NF), jnp.float32),
        mesh=_mesh(),
        scratch_types=[
            pltpu.VMEM((N_CHUNKS, CHUNK), jnp.int32),
            pltpu.VMEM((2, CHUNK, NF), jnp.float32),
            pltpu.SemaphoreType.DMA((2,)),
            pltpu.SemaphoreType.DMA((2,)),
        ],
    )
    def gather(nfeat_hbm, src_hbm, out_hbm, idx_all, rows_v, gsem, wsem):
        wid = lax.axis_index("s") * 2 + lax.axis_index("c")
        base = wid * EPW
        pltpu.sync_copy(src_hbm.at[wid], idx_all)
        pltpu.async_copy(nfeat_hbm.at[idx_all.at[0]], rows_v.at[0], gsem.at[0])

        def outer(t, carry):
            for b in range(2):
                j = 2 * t + b
                nb = 1 - b
                @pl.when(j < N_CHUNKS - 1)
                def _():
                    @pl.when(j >= 1)
                    def _():
                        pltpu.make_async_copy(
                            rows_v.at[nb],
                            out_hbm.at[pl.ds(base, CHUNK)],
                            wsem.at[nb],
                        ).wait()
                    pltpu.async_copy(
                        nfeat_hbm.at[idx_all.at[j + 1]], rows_v.at[nb], gsem.at[nb]
                    )

                pltpu.make_async_copy(
                    nfeat_hbm.at[idx_all.at[j]], rows_v.at[b], gsem.at[b]
                ).wait()
                pltpu.async_copy(
                    rows_v.at[b], out_hbm.at[pl.ds(base + j * CHUNK, CHUNK)], wsem.at[b]
                )
            return carry

        lax.fori_loop(0, N_CHUNKS // 2, outer, 0)
        for b in range(2):
            pltpu.make_async_copy(
                rows_v.at[b], out_hbm.at[pl.ds(base, CHUNK)], wsem.at[b]
            ).wait()

    return gather


def _sc_gather(nfeat, src3):
    return _sc_gather_fn()(nfeat, src3)


@functools.cache
def _sc_scatter_fn():
    @functools.partial(
        pl.kernel,
        out_type=jax.ShapeDtypeStruct((2, NACC, 128), jnp.float32),
        mesh=_mesh(),
        scratch_types=[
            pltpu.VMEM_SHARED((NACC, 128), jnp.float32),
            pltpu.VMEM((N_CHUNKS, CHUNK), jnp.int32),
            pltpu.VMEM((2, CHUNK, 128), jnp.float32),
            pltpu.SemaphoreType.DMA((2,)),
        ],
    )
    def scatter(msg_hbm, dst_hbm, zeros_hbm, out_hbm, acc_sh, idx_all, mrows_v, msem):
        cid = lax.axis_index("c")
        sid = lax.axis_index("s")
        wid = sid * 2 + cid
        r0 = sid * _RPT

        def zbody(k, carry):
            rr = r0 + k * _RCHUNK
            pltpu.sync_copy(zeros_hbm.at[pl.ds(rr, _RCHUNK)], mrows_v.at[0, pl.ds(0, _RCHUNK)])
            pltpu.sync_copy(mrows_v.at[0, pl.ds(0, _RCHUNK)], acc_sh.at[pl.ds(rr, _RCHUNK)])
            return carry

        lax.fori_loop(0, _RN, zbody, 0)
        pltpu.sync_copy(dst_hbm.at[wid], idx_all)
        plsc.subcore_barrier()

        base = wid * EPW
        pltpu.async_copy(msg_hbm.at[pl.ds(base, CHUNK)], mrows_v.at[0], msem.at[0])

        def body(t, carry):
            for b in range(2):
                j = 2 * t + b
                nb = 1 - b

                @pl.when(j < N_CHUNKS - 1)
                def _():
                    pltpu.async_copy(
                        msg_hbm.at[pl.ds(base + (j + 1) * CHUNK, CHUNK)],
                        mrows_v.at[nb],
                        msem.at[nb],
                    )

                pltpu.make_async_copy(
                    msg_hbm.at[pl.ds(base, CHUNK)], mrows_v.at[b], msem.at[b]
                ).wait()
                pltpu.sync_copy(mrows_v.at[b], acc_sh.at[idx_all.at[j]], add=True)
            return carry

        lax.fori_loop(0, N_CHUNKS // 2, body, 0)
        plsc.subcore_barrier()

        def obody(k, carry):
            rr = r0 + k * _RCHUNK
            pltpu.sync_copy(acc_sh.at[pl.ds(rr, _RCHUNK)], mrows_v.at[0, pl.ds(0, _RCHUNK)])
            pltpu.sync_copy(mrows_v.at[0, pl.ds(0, _RCHUNK)], out_hbm.at[cid, pl.ds(rr, _RCHUNK)])
            return carry

        lax.fori_loop(0, _RN, obody, 0)

    return scatter


def _sc_scatter(msg, dst3, zeros):
    return _sc_scatter_fn()(msg, dst3, zeros)


def _msg_body(srch_ref, ef_ref, acat_ref, out_ref):
    sf = jnp.dot(
        srch_ref[...].astype(jnp.bfloat16),
        acat_ref[...],
        preferred_element_type=jnp.float32,
    )
    ef = ef_ref[...]
    acc = ef[:, 0:1] * sf[:, 0:OF]
    for f in range(1, EF + 1):
        acc = acc + ef[:, f : f + 1] * sf[:, f * OF : (f + 1) * OF]
    out_ref[...] = jnp.concatenate(
        [acc * ef[:, EF + 1 : EF + 2], acc * ef[:, EF + 2 : EF + 3]], axis=1
    )


def _tc_msg(src_h, efeat_ext, acat):
    grid = E_PAD // BLK
    return pl.pallas_call(
        _msg_body,
        grid=(grid,),
        in_specs=[
            pl.BlockSpec((BLK, NF), lambda i: (i, 0)),
            pl.BlockSpec((BLK, 32), lambda i: (i, 0)),
            pl.BlockSpec((NF, (EF + 1) * OF), lambda i: (0, 0)),
        ],
        out_specs=pl.BlockSpec((BLK, 2 * OF), lambda i: (i, 0)),
        out_shape=jax.ShapeDtypeStruct((E_PAD, 2 * OF), jnp.float32),
    )(src_h, efeat_ext, acat)


def _head_body(parts_ref, eps_ref, w1t_ref, b1_ref, g1_ref, be1_ref,
               w2t_ref, b2_ref, g2_ref, be2_ref, out_ref):
    agg = (1.0 + eps_ref[0, 0]) * (parts_ref[0] + parts_ref[1])
    x = jnp.dot(agg, w1t_ref[...], preferred_element_type=jnp.float32) + b1_ref[...]
    m = jnp.mean(x, axis=0, keepdims=True)
    v = jnp.mean((x - m) ** 2, axis=0, keepdims=True)
    x = g1_ref[...] * (x - m) * lax.rsqrt(v + 1e-5) + be1_ref[...]
    x = jnp.maximum(x, 0.0)
    x = jnp.dot(x, w2t_ref[...], preferred_element_type=jnp.float32) + b2_ref[...]
    m2 = jnp.mean(x, axis=0, keepdims=True)
    v2 = jnp.mean((x - m2) ** 2, axis=0, keepdims=True)
    x = g2_ref[...] * (x - m2) * lax.rsqrt(v2 + 1e-5) + be2_ref[...]
    out_ref[...] = jnp.where(x >= 0.0, x, 0.01 * x)


def _tc_head(parts, eps, w1t, b1, g1, be1, w2t, b2, g2, be2):
    return pl.pallas_call(
        _head_body,
        in_specs=[
            pl.BlockSpec(memory_space=pltpu.VMEM),
            pl.BlockSpec(memory_space=pltpu.SMEM),
        ] + [pl.BlockSpec(memory_space=pltpu.VMEM)] * 8,
        out_specs=pl.BlockSpec(memory_space=pltpu.VMEM),
        out_shape=jax.ShapeDtypeStruct((N, OF), jnp.float32),
    )(parts, eps, w1t, b1, g1, be1, w2t, b2, g2, be2)


def kernel(nfeat, efeat, edge_index, eps, A, Ab, W1, b1, g1, be1, W2, b2, g2, be2):
    src = edge_index[0]
    dst = edge_index[1]
    pad = E_PAD - E
    src3 = jnp.concatenate([src, jnp.zeros((pad,), jnp.int32)]).reshape(
        NW, N_CHUNKS, CHUNK
    )
    dst_pad = jnp.concatenate([dst, jnp.zeros((pad,), jnp.int32)])
    dst3 = (dst_pad // 2).reshape(NW, N_CHUNKS, CHUNK)
    parity = (dst % 2).astype(jnp.float32)[:, None]
    ef_ext = jnp.concatenate(
        [efeat, jnp.ones((E, 1), jnp.float32), 1.0 - parity, parity], axis=1
    )
    ef_ext = jnp.pad(ef_ext, ((0, pad), (0, 32 - (EF + 3))))
    A3 = A.reshape(EF, NF, OF)
    acat = jnp.concatenate(
        [A3.transpose(1, 0, 2).reshape(NF, EF * OF), Ab.reshape(NF, OF)], axis=1
    ).astype(jnp.bfloat16)

    src_h = _sc_gather(nfeat, src3)
    msg = _tc_msg(src_h, ef_ext, acat)
    zeros = jnp.zeros((NACC, 128), jnp.float32)
    parts = _sc_scatter(msg, dst3, zeros)
    parts = parts.reshape(2, 2 * NACC, OF)[:, :N, :]
    return _tc_head(
        parts,
        eps.reshape(1, 1),
        W1.T, b1.reshape(1, H), g1.reshape(1, H), be1.reshape(1, H),
        W2.T, b2.reshape(1, OF), g2.reshape(1, OF), be2.reshape(1, OF),
    )


# speedup vs baseline: 1.4994x; 1.4757x over previous
import functools

import jax
import jax.numpy as jnp
from jax import lax
from jax.experimental import pallas as pl
from jax.experimental.pallas import tpu as pltpu
from jax.experimental.pallas import tpu_sc as plsc

N = 10000
E = 160000
NF = 128
OF = 64
EF = 16
H = 64

NW = 32
CHUNK = 128
EPW = 5120
E_PAD = NW * EPW
N_CHUNKS = EPW // CHUNK
BLK = 512

NACC = 5120
_RPT = NACC // 16
_RCHUNK = 64
_RN = _RPT // _RCHUNK


def _mesh():
    return plsc.VectorSubcoreMesh(core_axis_name="c", subcore_axis_name="s")


@functools.cache
def _sc_gather_fn():
    @functools.partial(
        pl.kernel,
        out_type=jax.ShapeDtypeStruct((E_PAD, NF), jnp.float32),
        mesh=_mesh(),
        scratch_types=[
            pltpu.VMEM((N_CHUNKS, CHUNK), jnp.int32),
            pltpu.VMEM((2, CHUNK, NF), jnp.float32),
            pltpu.SemaphoreType.DMA((2,)),
            pltpu.SemaphoreType.DMA((2,)),
        ],
    )
    def gather(nfeat_hbm, src_hbm, out_hbm, idx_all, rows_v, gsem, wsem):
        wid = lax.axis_index("s") * 2 + lax.axis_index("c")
        base = wid * EPW
        pltpu.sync_copy(src_hbm.at[wid], idx_all)
        pltpu.async_copy(nfeat_hbm.at[idx_all.at[0]], rows_v.at[0], gsem.at[0])

        def outer(t, carry):
            for b in range(2):
                j = 2 * t + b
                nb = 1 - b
                @pl.when(j < N_CHUNKS - 1)
                def _():
                    @pl.when(j >= 1)
                    def _():
                        pltpu.make_async_copy(
                            rows_v.at[nb],
                            out_hbm.at[pl.ds(base, CHUNK)],
                            wsem.at[nb],
                        ).wait()
                    pltpu.async_copy(
                        nfeat_hbm.at[idx_all.at[j + 1]], rows_v.at[nb], gsem.at[nb]
                    )

                pltpu.make_async_copy(
                    nfeat_hbm.at[idx_all.at[j]], rows_v.at[b], gsem.at[b]
                ).wait()
                pltpu.async_copy(
                    rows_v.at[b], out_hbm.at[pl.ds(base + j * CHUNK, CHUNK)], wsem.at[b]
                )
            return carry

        lax.fori_loop(0, N_CHUNKS // 2, outer, 0)
        for b in range(2):
            pltpu.make_async_copy(
                rows_v.at[b], out_hbm.at[pl.ds(base, CHUNK)], wsem.at[b]
            ).wait()

    return gather


def _sc_gather(nfeat, src3):
    return _sc_gather_fn()(nfeat, src3)


@functools.cache
def _sc_scatter_fn():
    @functools.partial(
        pl.kernel,
        out_type=jax.ShapeDtypeStruct((2, NACC, 128), jnp.float32),
        mesh=_mesh(),
        scratch_types=[
            pltpu.VMEM_SHARED((NACC, 128), jnp.float32),
            pltpu.VMEM((N_CHUNKS, CHUNK), jnp.int32),
            pltpu.VMEM((2, CHUNK, 128), jnp.float32),
            pltpu.SemaphoreType.DMA((2,)),
        ],
    )
    def scatter(msg_hbm, dst_hbm, zeros_hbm, out_hbm, acc_sh, idx_all, mrows_v, msem):
        cid = lax.axis_index("c")
        sid = lax.axis_index("s")
        wid = sid * 2 + cid
        r0 = sid * _RPT

        def zbody(k, carry):
            rr = r0 + k * _RCHUNK
            pltpu.sync_copy(zeros_hbm.at[pl.ds(rr, _RCHUNK)], mrows_v.at[0, pl.ds(0, _RCHUNK)])
            pltpu.sync_copy(mrows_v.at[0, pl.ds(0, _RCHUNK)], acc_sh.at[pl.ds(rr, _RCHUNK)])
            return carry

        lax.fori_loop(0, _RN, zbody, 0)
        pltpu.sync_copy(dst_hbm.at[wid], idx_all)
        plsc.subcore_barrier()

        base = wid * EPW
        pltpu.async_copy(msg_hbm.at[pl.ds(base, CHUNK)], mrows_v.at[0], msem.at[0])

        def body(t, carry):
            for b in range(2):
                j = 2 * t + b
                nb = 1 - b

                @pl.when(j < N_CHUNKS - 1)
                def _():
                    pltpu.async_copy(
                        msg_hbm.at[pl.ds(base + (j + 1) * CHUNK, CHUNK)],
                        mrows_v.at[nb],
                        msem.at[nb],
                    )

                pltpu.make_async_copy(
                    msg_hbm.at[pl.ds(base, CHUNK)], mrows_v.at[b], msem.at[b]
                ).wait()
                pltpu.sync_copy(mrows_v.at[b], acc_sh.at[idx_all.at[j]], add=True)
            return carry

        lax.fori_loop(0, N_CHUNKS // 2, body, 0)
        plsc.subcore_barrier()

        def obody(k, carry):
            rr = r0 + k * _RCHUNK
            pltpu.sync_copy(acc_sh.at[pl.ds(rr, _RCHUNK)], mrows_v.at[0, pl.ds(0, _RCHUNK)])
            pltpu.sync_copy(mrows_v.at[0, pl.ds(0, _RCHUNK)], out_hbm.at[cid, pl.ds(rr, _RCHUNK)])
            return carry

        lax.fori_loop(0, _RN, obody, 0)

    return scatter


def _sc_scatter(msg, dst3, zeros):
    return _sc_scatter_fn()(msg, dst3, zeros)


_NB = (EF + 2) // 2
_SFW = _NB * 128
_ESW = _SFW + 128


def _build_bsel():
    import numpy as np

    b = np.zeros((32, _ESW), np.float32)
    for f in range(EF + 1):
        c0 = 128 * (f // 2) + OF * (f % 2)
        b[f, c0 : c0 + OF] = 1.0
    b[EF + 1, _SFW : _SFW + OF] = 1.0
    b[EF + 2, _SFW + OF : _SFW + 2 * OF] = 1.0
    return jnp.asarray(b, jnp.float32)


def _msg_body(srch_ref, ef_ref, acat_ref, bsel_ref, out_ref):
    sf = jnp.dot(
        srch_ref[...].astype(jnp.bfloat16),
        acat_ref[...],
        preferred_element_type=jnp.float32,
    )
    efsel = jnp.dot(ef_ref[...], bsel_ref[...], preferred_element_type=jnp.float32)
    acc = efsel[:, 0:128] * sf[:, 0:128]
    for k in range(1, _NB):
        acc = acc + efsel[:, k * 128 : (k + 1) * 128] * sf[:, k * 128 : (k + 1) * 128]
    m2 = acc + jnp.concatenate([acc[:, OF:], acc[:, :OF]], axis=1)
    out_ref[...] = m2 * efsel[:, _SFW:_ESW]


def _tc_msg(src_h, efeat_ext, acat):
    grid = E_PAD // BLK
    return pl.pallas_call(
        _msg_body,
        grid=(grid,),
        in_specs=[
            pl.BlockSpec((BLK, NF), lambda i: (i, 0)),
            pl.BlockSpec((BLK, 32), lambda i: (i, 0)),
            pl.BlockSpec((NF, _SFW), lambda i: (0, 0)),
            pl.BlockSpec((32, _ESW), lambda i: (0, 0)),
        ],
        out_specs=pl.BlockSpec((BLK, 2 * OF), lambda i: (i, 0)),
        out_shape=jax.ShapeDtypeStruct((E_PAD, 2 * OF), jnp.float32),
    )(src_h, efeat_ext, acat, _build_bsel())


def _head_body(parts_ref, eps_ref, w1t_ref, b1_ref, g1_ref, be1_ref,
               w2t_ref, b2_ref, g2_ref, be2_ref, out_ref):
    agg = (1.0 + eps_ref[0, 0]) * (parts_ref[0] + parts_ref[1])
    x = jnp.dot(agg, w1t_ref[...], preferred_element_type=jnp.float32) + b1_ref[...]
    m = jnp.mean(x, axis=0, keepdims=True)
    v = jnp.mean((x - m) ** 2, axis=0, keepdims=True)
    x = g1_ref[...] * (x - m) * lax.rsqrt(v + 1e-5) + be1_ref[...]
    x = jnp.maximum(x, 0.0)
    x = jnp.dot(x, w2t_ref[...], preferred_element_type=jnp.float32) + b2_ref[...]
    m2 = jnp.mean(x, axis=0, keepdims=True)
    v2 = jnp.mean((x - m2) ** 2, axis=0, keepdims=True)
    x = g2_ref[...] * (x - m2) * lax.rsqrt(v2 + 1e-5) + be2_ref[...]
    out_ref[...] = jnp.where(x >= 0.0, x, 0.01 * x)


def _tc_head(parts, eps, w1t, b1, g1, be1, w2t, b2, g2, be2):
    return pl.pallas_call(
        _head_body,
        in_specs=[
            pl.BlockSpec(memory_space=pltpu.VMEM),
            pl.BlockSpec(memory_space=pltpu.SMEM),
        ] + [pl.BlockSpec(memory_space=pltpu.VMEM)] * 8,
        out_specs=pl.BlockSpec(memory_space=pltpu.VMEM),
        out_shape=jax.ShapeDtypeStruct((N, OF), jnp.float32),
    )(parts, eps, w1t, b1, g1, be1, w2t, b2, g2, be2)


def kernel(nfeat, efeat, edge_index, eps, A, Ab, W1, b1, g1, be1, W2, b2, g2, be2):
    src = edge_index[0]
    dst = edge_index[1]
    pad = E_PAD - E
    src3 = jnp.concatenate([src, jnp.zeros((pad,), jnp.int32)]).reshape(
        NW, N_CHUNKS, CHUNK
    )
    dst_pad = jnp.concatenate([dst, jnp.zeros((pad,), jnp.int32)])
    dst3 = (dst_pad // 2).reshape(NW, N_CHUNKS, CHUNK)
    parity = (dst % 2).astype(jnp.float32)[:, None]
    ef_ext = jnp.concatenate(
        [efeat, jnp.ones((E, 1), jnp.float32), 1.0 - parity, parity], axis=1
    )
    ef_ext = jnp.pad(ef_ext, ((0, pad), (0, 32 - (EF + 3))))
    A3 = A.reshape(EF, NF, OF)
    planes = jnp.concatenate(
        [A3, Ab.reshape(1, NF, OF), jnp.zeros((1, NF, OF), jnp.float32)], axis=0
    )
    acat = (
        planes.reshape(_NB, 2, NF, OF)
        .transpose(2, 0, 1, 3)
        .reshape(NF, _SFW)
        .astype(jnp.bfloat16)
    )

    src_h = _sc_gather(nfeat, src3)
    msg = _tc_msg(src_h, ef_ext, acat)
    zeros = jnp.zeros((NACC, 128), jnp.float32)
    parts = _sc_scatter(msg, dst3, zeros)
    parts = parts.reshape(2, 2 * NACC, OF)[:, :N, :]
    return _tc_head(
        parts,
        eps.reshape(1, 1),
        W1.T, b1.reshape(1, H), g1.reshape(1, H), be1.reshape(1, H),
        W2.T, b2.reshape(1, OF), g2.reshape(1, OF), be2.reshape(1, OF),
    )


# speedup vs baseline: 1.5802x; 1.0539x over previous
import functools

import jax
import jax.numpy as jnp
from jax import lax
from jax.experimental import pallas as pl
from jax.experimental.pallas import tpu as pltpu
from jax.experimental.pallas import tpu_sc as plsc

N = 10000
E = 160000
NF = 128
OF = 64
EF = 16
H = 64

NW = 32
CHUNK = 128
EPW = 5120
E_PAD = NW * EPW
N_CHUNKS = EPW // CHUNK
BLK = 512

NACC = 5120
_RPT = NACC // 16
_RCHUNK = 64
_RN = _RPT // _RCHUNK


def _mesh():
    return plsc.VectorSubcoreMesh(core_axis_name="c", subcore_axis_name="s")


@functools.cache
def _sc_gather_fn(nchunks):
    epw = nchunks * CHUNK
    @functools.partial(
        pl.kernel,
        out_type=jax.ShapeDtypeStruct((NW * epw, NF), jnp.float32),
        mesh=_mesh(),
        scratch_types=[
            pltpu.VMEM((nchunks, CHUNK), jnp.int32),
            pltpu.VMEM((2, CHUNK, NF), jnp.float32),
            pltpu.SemaphoreType.DMA((2,)),
            pltpu.SemaphoreType.DMA((2,)),
        ],
    )
    def gather(nfeat_hbm, src_hbm, out_hbm, idx_all, rows_v, gsem, wsem):
        wid = lax.axis_index("s") * 2 + lax.axis_index("c")
        base = wid * epw
        pltpu.sync_copy(src_hbm.at[wid], idx_all)
        pltpu.async_copy(nfeat_hbm.at[idx_all.at[0]], rows_v.at[0], gsem.at[0])

        def outer(t, carry):
            for b in range(2):
                j = 2 * t + b
                nb = 1 - b
                @pl.when(j < nchunks - 1)
                def _():
                    @pl.when(j >= 1)
                    def _():
                        pltpu.make_async_copy(
                            rows_v.at[nb],
                            out_hbm.at[pl.ds(base, CHUNK)],
                            wsem.at[nb],
                        ).wait()
                    pltpu.async_copy(
                        nfeat_hbm.at[idx_all.at[j + 1]], rows_v.at[nb], gsem.at[nb]
                    )

                pltpu.make_async_copy(
                    nfeat_hbm.at[idx_all.at[j]], rows_v.at[b], gsem.at[b]
                ).wait()
                pltpu.async_copy(
                    rows_v.at[b], out_hbm.at[pl.ds(base + j * CHUNK, CHUNK)], wsem.at[b]
                )
            return carry

        lax.fori_loop(0, nchunks // 2, outer, 0)
        for b in range(2):
            pltpu.make_async_copy(
                rows_v.at[b], out_hbm.at[pl.ds(base, CHUNK)], wsem.at[b]
            ).wait()

    return gather


def _sc_gather(nfeat, src3):
    return _sc_gather_fn(src3.shape[1])(nfeat, src3)


@functools.cache
def _sc_scatter_fn():
    @functools.partial(
        pl.kernel,
        out_type=jax.ShapeDtypeStruct((2, NACC, 128), jnp.float32),
        mesh=_mesh(),
        scratch_types=[
            pltpu.VMEM_SHARED((NACC, 128), jnp.float32),
            pltpu.VMEM((N_CHUNKS, CHUNK), jnp.int32),
            pltpu.VMEM((2, CHUNK, 128), jnp.float32),
            pltpu.SemaphoreType.DMA((2,)),
        ],
    )
    def scatter(msg1_hbm, msg2_hbm, dst_hbm, zeros_hbm, out_hbm, acc_sh, idx_all, mrows_v, msem):
        cid = lax.axis_index("c")
        sid = lax.axis_index("s")
        wid = sid * 2 + cid
        r0 = sid * _RPT

        def zbody(k, carry):
            rr = r0 + k * _RCHUNK
            pltpu.sync_copy(zeros_hbm.at[pl.ds(rr, _RCHUNK)], mrows_v.at[0, pl.ds(0, _RCHUNK)])
            pltpu.sync_copy(mrows_v.at[0, pl.ds(0, _RCHUNK)], acc_sh.at[pl.ds(rr, _RCHUNK)])
            return carry

        lax.fori_loop(0, _RN, zbody, 0)
        pltpu.sync_copy(dst_hbm.at[wid], idx_all)
        plsc.subcore_barrier()

        def run_half(msg_hbm, base):
            pltpu.async_copy(msg_hbm.at[pl.ds(base, CHUNK)], mrows_v.at[0], msem.at[0])

            def body(t, carry):
                for b in range(2):
                    j = 2 * t + b
                    nb = 1 - b

                    @pl.when(j < N_CHUNKS - 1)
                    def _():
                        pltpu.async_copy(
                            msg_hbm.at[pl.ds(base + (j + 1) * CHUNK, CHUNK)],
                            mrows_v.at[nb],
                            msem.at[nb],
                        )

                    pltpu.make_async_copy(
                        msg_hbm.at[pl.ds(base, CHUNK)], mrows_v.at[b], msem.at[b]
                    ).wait()
                    pltpu.sync_copy(mrows_v.at[b], acc_sh.at[idx_all.at[j]], add=True)
                return carry

            lax.fori_loop(0, N_CHUNKS // 2, body, 0)

        @pl.when(sid < 8)
        def _():
            run_half(msg1_hbm, wid * EPW)

        @pl.when(sid >= 8)
        def _():
            run_half(msg2_hbm, wid * EPW - E_PAD // 2)
        plsc.subcore_barrier()

        def obody(k, carry):
            rr = r0 + k * _RCHUNK
            pltpu.sync_copy(acc_sh.at[pl.ds(rr, _RCHUNK)], mrows_v.at[0, pl.ds(0, _RCHUNK)])
            pltpu.sync_copy(mrows_v.at[0, pl.ds(0, _RCHUNK)], out_hbm.at[cid, pl.ds(rr, _RCHUNK)])
            return carry

        lax.fori_loop(0, _RN, obody, 0)

    return scatter


def _sc_scatter(msg1, msg2, dst3, zeros):
    return _sc_scatter_fn()(msg1, msg2, dst3, zeros)


_NB = (EF + 2) // 2
_SFW = _NB * 128
_ESW = _SFW + 128


def _build_bsel():
    import numpy as np

    b = np.zeros((32, _ESW), np.float32)
    for f in range(EF + 1):
        c0 = 128 * (f // 2) + OF * (f % 2)
        b[f, c0 : c0 + OF] = 1.0
    b[EF + 1, _SFW : _SFW + OF] = 1.0
    b[EF + 2, _SFW + OF : _SFW + 2 * OF] = 1.0
    return jnp.asarray(b, jnp.float32)


def _msg_body(srch_ref, ef_ref, acat_ref, bsel_ref, out_ref):
    sf = jnp.dot(
        srch_ref[...].astype(jnp.bfloat16),
        acat_ref[...],
        preferred_element_type=jnp.float32,
    )
    efsel = jnp.dot(ef_ref[...], bsel_ref[...], preferred_element_type=jnp.float32)
    acc = efsel[:, 0:128] * sf[:, 0:128]
    for k in range(1, _NB):
        acc = acc + efsel[:, k * 128 : (k + 1) * 128] * sf[:, k * 128 : (k + 1) * 128]
    m2 = acc + jnp.concatenate([acc[:, OF:], acc[:, :OF]], axis=1)
    out_ref[...] = m2 * efsel[:, _SFW:_ESW]


def _tc_msg(src_h, efeat_ext, acat):
    grid = src_h.shape[0] // BLK
    return pl.pallas_call(
        _msg_body,
        grid=(grid,),
        in_specs=[
            pl.BlockSpec((BLK, NF), lambda i: (i, 0)),
            pl.BlockSpec((BLK, 32), lambda i: (i, 0)),
            pl.BlockSpec((NF, _SFW), lambda i: (0, 0)),
            pl.BlockSpec((32, _ESW), lambda i: (0, 0)),
        ],
        out_specs=pl.BlockSpec((BLK, 2 * OF), lambda i: (i, 0)),
        out_shape=jax.ShapeDtypeStruct((src_h.shape[0], 2 * OF), jnp.float32),
    )(src_h, efeat_ext, acat, _build_bsel())


def _head_body(parts_ref, eps_ref, w1t_ref, b1_ref, g1_ref, be1_ref,
               w2t_ref, b2_ref, g2_ref, be2_ref, out_ref):
    agg = (1.0 + eps_ref[0, 0]) * (parts_ref[0] + parts_ref[1])
    x = jnp.dot(agg, w1t_ref[...], preferred_element_type=jnp.float32) + b1_ref[...]
    m = jnp.mean(x, axis=0, keepdims=True)
    v = jnp.mean((x - m) ** 2, axis=0, keepdims=True)
    x = g1_ref[...] * (x - m) * lax.rsqrt(v + 1e-5) + be1_ref[...]
    x = jnp.maximum(x, 0.0)
    x = jnp.dot(x, w2t_ref[...], preferred_element_type=jnp.float32) + b2_ref[...]
    m2 = jnp.mean(x, axis=0, keepdims=True)
    v2 = jnp.mean((x - m2) ** 2, axis=0, keepdims=True)
    x = g2_ref[...] * (x - m2) * lax.rsqrt(v2 + 1e-5) + be2_ref[...]
    out_ref[...] = jnp.where(x >= 0.0, x, 0.01 * x)


def _tc_head(parts, eps, w1t, b1, g1, be1, w2t, b2, g2, be2):
    return pl.pallas_call(
        _head_body,
        in_specs=[
            pl.BlockSpec(memory_space=pltpu.VMEM),
            pl.BlockSpec(memory_space=pltpu.SMEM),
        ] + [pl.BlockSpec(memory_space=pltpu.VMEM)] * 8,
        out_specs=pl.BlockSpec(memory_space=pltpu.VMEM),
        out_shape=jax.ShapeDtypeStruct((N, OF), jnp.float32),
    )(parts, eps, w1t, b1, g1, be1, w2t, b2, g2, be2)


def kernel(nfeat, efeat, edge_index, eps, A, Ab, W1, b1, g1, be1, W2, b2, g2, be2):
    src = edge_index[0]
    dst = edge_index[1]
    pad = E_PAD - E
    src3 = jnp.concatenate([src, jnp.zeros((pad,), jnp.int32)]).reshape(
        NW, N_CHUNKS, CHUNK
    )
    dst_pad = jnp.concatenate([dst, jnp.zeros((pad,), jnp.int32)])
    dst3 = (dst_pad // 2).reshape(NW, N_CHUNKS, CHUNK)
    parity = (dst % 2).astype(jnp.float32)[:, None]
    ef_ext = jnp.concatenate(
        [efeat, jnp.ones((E, 1), jnp.float32), 1.0 - parity, parity], axis=1
    )
    ef_ext = jnp.pad(ef_ext, ((0, pad), (0, 32 - (EF + 3))))
    A3 = A.reshape(EF, NF, OF)
    planes = jnp.concatenate(
        [A3, Ab.reshape(1, NF, OF), jnp.zeros((1, NF, OF), jnp.float32)], axis=0
    )
    acat = (
        planes.reshape(_NB, 2, NF, OF)
        .transpose(2, 0, 1, 3)
        .reshape(NF, _SFW)
        .astype(jnp.bfloat16)
    )

    half = E_PAD // 2
    srcA = src3.reshape(E_PAD)[:half].reshape(NW, N_CHUNKS // 2, CHUNK)
    srcB = src3.reshape(E_PAD)[half:].reshape(NW, N_CHUNKS // 2, CHUNK)
    src_hA = _sc_gather(nfeat, srcA)
    src_hB = _sc_gather(nfeat, srcB)
    msgA = _tc_msg(src_hA, ef_ext[:half], acat)
    msgB = _tc_msg(src_hB, ef_ext[half:], acat)
    zeros = jnp.zeros((NACC, 128), jnp.float32)
    parts = _sc_scatter(msgA, msgB, dst3, zeros)
    parts = parts.reshape(2, 2 * NACC, OF)[:, :N, :]
    return _tc_head(
        parts,
        eps.reshape(1, 1),
        W1.T, b1.reshape(1, H), g1.reshape(1, H), be1.reshape(1, H),
        W2.T, b2.reshape(1, OF), g2.reshape(1, OF), be2.reshape(1, OF),
    )
